# baseline (speedup 1.0000x reference)
import functools

import jax
import jax.numpy as jnp
from jax.experimental import pallas as pl
from jax.experimental.pallas import tpu as pltpu

_TEMPERATURE = 0.07
_CHUNK = 2048


def _body(n_rows, n_steps, p_sz, p_nsz,
          w_ref, h_ref, emb_ref, psz_ref, pnsz_ref, out_ref, acc_ref, cnt_ref):
    c = pl.program_id(0)

    @pl.when(c == 0)
    def _init():
        acc_ref[...] = jnp.zeros_like(acc_ref)
        cnt_ref[0] = jnp.float32(0.0)

    wb = w_ref[...]
    hb = h_ref[...]
    acc_ref[...] += jax.lax.dot_general(
        wb, hb, (((0,), (0,)), ((), ())), preferred_element_type=jnp.float32)
    cnt_ref[0] += jnp.sum(wb[:, 1:2])

    @pl.when(c == n_steps - 1)
    def _epilogue():
        s_all = acc_ref[0:1, :]
        s_y = acc_ref[1:2, :]
        n1 = cnt_ref[0]
        n0 = jnp.float32(n_rows) - n1
        m_sz = s_y / jnp.maximum(n1, 1.0)
        m_nsz = (s_all - s_y) / jnp.maximum(n0, 1.0)
        m_sz = m_sz / jnp.maximum(jnp.sqrt(jnp.sum(m_sz * m_sz)), 1e-12)
        m_nsz = m_nsz / jnp.maximum(jnp.sqrt(jnp.sum(m_nsz * m_nsz)), 1e-12)

        emb = emb_ref[...]
        z_sz = jax.lax.dot_general(
            m_sz, emb, (((1,), (1,)), ((), ())),
            preferred_element_type=jnp.float32) / _TEMPERATURE
        z_nsz = jax.lax.dot_general(
            m_nsz, emb, (((1,), (1,)), ((), ())),
            preferred_element_type=jnp.float32) / _TEMPERATURE

        sim_sz = jnp.exp(z_sz)
        sim_nsz = jnp.exp(z_nsz)
        lanes = jax.lax.broadcasted_iota(jnp.int32, z_sz.shape, 1)

        mask_sz = jnp.zeros(z_sz.shape, dtype=jnp.bool_)
        zsum_sz = jnp.float32(0.0)
        for i in range(p_sz):
            hit = lanes == psz_ref[i]
            mask_sz = jnp.logical_or(mask_sz, hit)
            zsum_sz += jnp.sum(jnp.where(hit, z_sz, 0.0))
        den_sz = jnp.sum(jnp.where(mask_sz, 0.0, sim_sz))
        loss_sz = jnp.log(den_sz) - zsum_sz / jnp.float32(p_sz)

        mask_nsz = jnp.zeros(z_nsz.shape, dtype=jnp.bool_)
        zsum_nsz = jnp.float32(0.0)
        for i in range(p_nsz):
            hit = lanes == pnsz_ref[i]
            mask_nsz = jnp.logical_or(mask_nsz, hit)
            zsum_nsz += jnp.sum(jnp.where(hit, z_nsz, 0.0))
        den_nsz = jnp.sum(jnp.where(mask_nsz, 0.0, sim_nsz))
        loss_nsz = jnp.log(den_nsz) - zsum_nsz / jnp.float32(p_nsz)

        out_ref[...] = jnp.broadcast_to(loss_sz + loss_nsz, (1, 1))


def kernel(hg, all_emb, det_labels, concept_labels, Psz_idx, Pnsz_idx):
    del concept_labels
    b, nsz, t, l = hg.shape
    n = b * nsz * t
    h2 = hg.reshape(n, l)
    yf = det_labels.reshape(n, 1).astype(jnp.float32)
    w = jnp.concatenate(
        [jnp.ones((n, 1), jnp.float32), yf, jnp.zeros((n, 6), jnp.float32)],
        axis=1)

    n_steps = n // _CHUNK
    p_sz = int(Psz_idx.shape[0])
    p_nsz = int(Pnsz_idx.shape[0])

    body = functools.partial(_body, n, n_steps, p_sz, p_nsz)

    out = pl.pallas_call(
        body,
        grid=(n_steps,),
        in_specs=[
            pl.BlockSpec((_CHUNK, 8), lambda c: (c, 0)),
            pl.BlockSpec((_CHUNK, l), lambda c: (c, 0)),
            pl.BlockSpec((all_emb.shape[0], l), lambda c: (0, 0)),
            pl.BlockSpec(memory_space=pltpu.SMEM),
            pl.BlockSpec(memory_space=pltpu.SMEM),
        ],
        out_specs=pl.BlockSpec((1, 1), lambda c: (0, 0)),
        out_shape=jax.ShapeDtypeStruct((1, 1), jnp.float32),
        scratch_shapes=[
            pltpu.VMEM((8, l), jnp.float32),
            pltpu.SMEM((1,), jnp.float32),
        ],
    )(w, h2, all_emb, Psz_idx, Pnsz_idx)
    return out[0, 0]


# speedup vs baseline: 1.0124x; 1.0124x over previous
import functools

import jax
import jax.numpy as jnp
from jax.experimental import pallas as pl
from jax.experimental.pallas import tpu as pltpu

_TEMPERATURE = 0.07
_CHUNK = 8192


def _body(n_rows, n_steps, p_sz, p_nsz,
          w_ref, h_ref, emb_ref, psz_ref, pnsz_ref, out_ref, acc_ref, cnt_ref):
    c = pl.program_id(0)

    @pl.when(c == 0)
    def _init():
        acc_ref[...] = jnp.zeros_like(acc_ref)
        cnt_ref[0] = jnp.float32(0.0)

    wb = w_ref[...]
    hb = h_ref[...]
    acc_ref[...] += jax.lax.dot_general(
        wb, hb, (((0,), (0,)), ((), ())), preferred_element_type=jnp.float32)
    cnt_ref[0] += jnp.sum(wb[:, 1:2])

    @pl.when(c == n_steps - 1)
    def _epilogue():
        s_all = acc_ref[0:1, :]
        s_y = acc_ref[1:2, :]
        n1 = cnt_ref[0]
        n0 = jnp.float32(n_rows) - n1
        m_sz = s_y / jnp.maximum(n1, 1.0)
        m_nsz = (s_all - s_y) / jnp.maximum(n0, 1.0)
        m_sz = m_sz / jnp.maximum(jnp.sqrt(jnp.sum(m_sz * m_sz)), 1e-12)
        m_nsz = m_nsz / jnp.maximum(jnp.sqrt(jnp.sum(m_nsz * m_nsz)), 1e-12)

        emb = emb_ref[...]
        z_sz = jax.lax.dot_general(
            m_sz, emb, (((1,), (1,)), ((), ())),
            preferred_element_type=jnp.float32) / _TEMPERATURE
        z_nsz = jax.lax.dot_general(
            m_nsz, emb, (((1,), (1,)), ((), ())),
            preferred_element_type=jnp.float32) / _TEMPERATURE

        sim_sz = jnp.exp(z_sz)
        sim_nsz = jnp.exp(z_nsz)
        lanes = jax.lax.broadcasted_iota(jnp.int32, z_sz.shape, 1)

        mask_sz = jnp.zeros(z_sz.shape, dtype=jnp.bool_)
        zsum_sz = jnp.float32(0.0)
        for i in range(p_sz):
            hit = lanes == psz_ref[i]
            mask_sz = jnp.logical_or(mask_sz, hit)
            zsum_sz += jnp.sum(jnp.where(hit, z_sz, 0.0))
        den_sz = jnp.sum(jnp.where(mask_sz, 0.0, sim_sz))
        loss_sz = jnp.log(den_sz) - zsum_sz / jnp.float32(p_sz)

        mask_nsz = jnp.zeros(z_nsz.shape, dtype=jnp.bool_)
        zsum_nsz = jnp.float32(0.0)
        for i in range(p_nsz):
            hit = lanes == pnsz_ref[i]
            mask_nsz = jnp.logical_or(mask_nsz, hit)
            zsum_nsz += jnp.sum(jnp.where(hit, z_nsz, 0.0))
        den_nsz = jnp.sum(jnp.where(mask_nsz, 0.0, sim_nsz))
        loss_nsz = jnp.log(den_nsz) - zsum_nsz / jnp.float32(p_nsz)

        out_ref[...] = jnp.broadcast_to(loss_sz + loss_nsz, (1, 1))


def kernel(hg, all_emb, det_labels, concept_labels, Psz_idx, Pnsz_idx):
    del concept_labels
    b, nsz, t, l = hg.shape
    n = b * nsz * t
    h2 = hg.reshape(n, l)
    yf = det_labels.reshape(n, 1).astype(jnp.float32)
    w = jnp.concatenate(
        [jnp.ones((n, 1), jnp.float32), yf, jnp.zeros((n, 6), jnp.float32)],
        axis=1)

    n_steps = n // _CHUNK
    p_sz = int(Psz_idx.shape[0])
    p_nsz = int(Pnsz_idx.shape[0])

    body = functools.partial(_body, n, n_steps, p_sz, p_nsz)

    out = pl.pallas_call(
        body,
        grid=(n_steps,),
        in_specs=[
            pl.BlockSpec((_CHUNK, 8), lambda c: (c, 0)),
            pl.BlockSpec((_CHUNK, l), lambda c: (c, 0)),
            pl.BlockSpec((all_emb.shape[0], l), lambda c: (0, 0)),
            pl.BlockSpec(memory_space=pltpu.SMEM),
            pl.BlockSpec(memory_space=pltpu.SMEM),
        ],
        out_specs=pl.BlockSpec((1, 1), lambda c: (0, 0)),
        out_shape=jax.ShapeDtypeStruct((1, 1), jnp.float32),
        scratch_shapes=[
            pltpu.VMEM((8, l), jnp.float32),
            pltpu.SMEM((1,), jnp.float32),
        ],
    )(w, h2, all_emb, Psz_idx, Pnsz_idx)
    return out[0, 0]


# speedup vs baseline: 1.0220x; 1.0095x over previous
import functools

import jax
import jax.numpy as jnp
from jax.experimental import pallas as pl
from jax.experimental.pallas import tpu as pltpu

_TEMPERATURE = 0.07
_CHUNK = 2048


def _body(n_rows, n_steps, p_sz, p_nsz,
          w_ref, h0_ref, h1_ref, h2_ref, h3_ref, emb_ref, psz_ref, pnsz_ref,
          out_ref, acc_ref, cnt_ref):
    c = pl.program_id(0)

    @pl.when(c == 0)
    def _init():
        acc_ref[...] = jnp.zeros_like(acc_ref)
        cnt_ref[0] = jnp.float32(0.0)

    wb = w_ref[...]
    acc = acc_ref[...]
    for i, h_ref in enumerate((h0_ref, h1_ref, h2_ref, h3_ref)):
        hb = h_ref[...]
        acc += jax.lax.dot_general(
            wb[i * _CHUNK:(i + 1) * _CHUNK, :], hb,
            (((0,), (0,)), ((), ())), preferred_element_type=jnp.float32)
    acc_ref[...] = acc
    cnt_ref[0] += jnp.sum(wb[:, 1:2])

    @pl.when(c == n_steps - 1)
    def _epilogue():
        s_all = acc_ref[0:1, :]
        s_y = acc_ref[1:2, :]
        n1 = cnt_ref[0]
        n0 = jnp.float32(n_rows) - n1
        m_sz = s_y / jnp.maximum(n1, 1.0)
        m_nsz = (s_all - s_y) / jnp.maximum(n0, 1.0)
        m_sz = m_sz / jnp.maximum(jnp.sqrt(jnp.sum(m_sz * m_sz)), 1e-12)
        m_nsz = m_nsz / jnp.maximum(jnp.sqrt(jnp.sum(m_nsz * m_nsz)), 1e-12)

        emb = emb_ref[...]
        z_sz = jax.lax.dot_general(
            m_sz, emb, (((1,), (1,)), ((), ())),
            preferred_element_type=jnp.float32) / _TEMPERATURE
        z_nsz = jax.lax.dot_general(
            m_nsz, emb, (((1,), (1,)), ((), ())),
            preferred_element_type=jnp.float32) / _TEMPERATURE

        sim_sz = jnp.exp(z_sz)
        sim_nsz = jnp.exp(z_nsz)
        lanes = jax.lax.broadcasted_iota(jnp.int32, z_sz.shape, 1)

        mask_sz = jnp.zeros(z_sz.shape, dtype=jnp.bool_)
        zsum_sz = jnp.float32(0.0)
        for i in range(p_sz):
            hit = lanes == psz_ref[i]
            mask_sz = jnp.logical_or(mask_sz, hit)
            zsum_sz += jnp.sum(jnp.where(hit, z_sz, 0.0))
        den_sz = jnp.sum(jnp.where(mask_sz, 0.0, sim_sz))
        loss_sz = jnp.log(den_sz) - zsum_sz / jnp.float32(p_sz)

        mask_nsz = jnp.zeros(z_nsz.shape, dtype=jnp.bool_)
        zsum_nsz = jnp.float32(0.0)
        for i in range(p_nsz):
            hit = lanes == pnsz_ref[i]
            mask_nsz = jnp.logical_or(mask_nsz, hit)
            zsum_nsz += jnp.sum(jnp.where(hit, z_nsz, 0.0))
        den_nsz = jnp.sum(jnp.where(mask_nsz, 0.0, sim_nsz))
        loss_nsz = jnp.log(den_nsz) - zsum_nsz / jnp.float32(p_nsz)

        out_ref[...] = jnp.broadcast_to(loss_sz + loss_nsz, (1, 1))


def kernel(hg, all_emb, det_labels, concept_labels, Psz_idx, Pnsz_idx):
    del concept_labels
    b, nsz, t, l = hg.shape
    n = b * nsz * t
    h2 = hg.reshape(n, l)
    yf = det_labels.reshape(n, 1).astype(jnp.float32)
    w = jnp.concatenate(
        [jnp.ones((n, 1), jnp.float32), yf, jnp.zeros((n, 6), jnp.float32)],
        axis=1)

    n_steps = n // (4 * _CHUNK)
    p_sz = int(Psz_idx.shape[0])
    p_nsz = int(Pnsz_idx.shape[0])

    body = functools.partial(_body, n, n_steps, p_sz, p_nsz)

    h_specs = [
        pl.BlockSpec((_CHUNK, l), functools.partial(lambda i, c: (4 * c + i, 0), i))
        for i in range(4)
    ]
    out = pl.pallas_call(
        body,
        grid=(n_steps,),
        in_specs=[
            pl.BlockSpec((4 * _CHUNK, 8), lambda c: (c, 0)),
            *h_specs,
            pl.BlockSpec((all_emb.shape[0], l), lambda c: (0, 0)),
            pl.BlockSpec(memory_space=pltpu.SMEM),
            pl.BlockSpec(memory_space=pltpu.SMEM),
        ],
        out_specs=pl.BlockSpec((1, 1), lambda c: (0, 0)),
        out_shape=jax.ShapeDtypeStruct((1, 1), jnp.float32),
        scratch_shapes=[
            pltpu.VMEM((8, l), jnp.float32),
            pltpu.SMEM((1,), jnp.float32),
        ],
    )(w, h2, h2, h2, h2, all_emb, Psz_idx, Pnsz_idx)
    return out[0, 0]


# speedup vs baseline: 1.0346x; 1.0123x over previous
import functools

import jax
import jax.numpy as jnp
from jax.experimental import pallas as pl
from jax.experimental.pallas import tpu as pltpu

_TEMPERATURE = 0.07
_CHUNK = 2048


def _body(n_rows, n_steps, p_sz, p_nsz,
          y_ref, h_ref, emb_ref, psz_ref, pnsz_ref,
          out_ref, acc_ref, cnt_ref):
    c = pl.program_id(0)

    @pl.when(c == 0)
    def _init():
        acc_ref[...] = jnp.zeros_like(acc_ref)
        cnt_ref[0] = jnp.float32(0.0)

    hb = h_ref[...]
    yb = y_ref[...]
    sel = jnp.where(yb > 0.5, hb, 0.0)
    h3 = hb.reshape(_CHUNK // 8, 8, 512)
    s3 = sel.reshape(_CHUNK // 8, 8, 512)
    acc_ref[0:8, :] += jnp.sum(h3, axis=0)
    acc_ref[8:16, :] += jnp.sum(s3, axis=0)
    cnt_ref[0] += jnp.sum(yb)

    @pl.when(c == n_steps - 1)
    def _epilogue():
        s_all = jnp.sum(acc_ref[0:8, :], axis=0, keepdims=True)
        s_y = jnp.sum(acc_ref[8:16, :], axis=0, keepdims=True)
        n1 = cnt_ref[0]
        n0 = jnp.float32(n_rows) - n1
        m_sz = s_y / jnp.maximum(n1, 1.0)
        m_nsz = (s_all - s_y) / jnp.maximum(n0, 1.0)
        m_sz = m_sz / jnp.maximum(jnp.sqrt(jnp.sum(m_sz * m_sz)), 1e-12)
        m_nsz = m_nsz / jnp.maximum(jnp.sqrt(jnp.sum(m_nsz * m_nsz)), 1e-12)

        emb = emb_ref[...]
        z_sz = jax.lax.dot_general(
            m_sz, emb, (((1,), (1,)), ((), ())),
            preferred_element_type=jnp.float32) / _TEMPERATURE
        z_nsz = jax.lax.dot_general(
            m_nsz, emb, (((1,), (1,)), ((), ())),
            preferred_element_type=jnp.float32) / _TEMPERATURE

        sim_sz = jnp.exp(z_sz)
        sim_nsz = jnp.exp(z_nsz)
        lanes = jax.lax.broadcasted_iota(jnp.int32, z_sz.shape, 1)

        mask_sz = jnp.zeros(z_sz.shape, dtype=jnp.bool_)
        zsum_sz = jnp.float32(0.0)
        for i in range(p_sz):
            hit = lanes == psz_ref[i]
            mask_sz = jnp.logical_or(mask_sz, hit)
            zsum_sz += jnp.sum(jnp.where(hit, z_sz, 0.0))
        den_sz = jnp.sum(jnp.where(mask_sz, 0.0, sim_sz))
        loss_sz = jnp.log(den_sz) - zsum_sz / jnp.float32(p_sz)

        mask_nsz = jnp.zeros(z_nsz.shape, dtype=jnp.bool_)
        zsum_nsz = jnp.float32(0.0)
        for i in range(p_nsz):
            hit = lanes == pnsz_ref[i]
            mask_nsz = jnp.logical_or(mask_nsz, hit)
            zsum_nsz += jnp.sum(jnp.where(hit, z_nsz, 0.0))
        den_nsz = jnp.sum(jnp.where(mask_nsz, 0.0, sim_nsz))
        loss_nsz = jnp.log(den_nsz) - zsum_nsz / jnp.float32(p_nsz)

        out_ref[...] = jnp.broadcast_to(loss_sz + loss_nsz, (1, 1))


def kernel(hg, all_emb, det_labels, concept_labels, Psz_idx, Pnsz_idx):
    del concept_labels
    b, nsz, t, l = hg.shape
    n = b * nsz * t
    h2 = hg.reshape(n, l)
    yf = det_labels.reshape(n, 1).astype(jnp.float32)

    n_steps = n // _CHUNK
    p_sz = int(Psz_idx.shape[0])
    p_nsz = int(Pnsz_idx.shape[0])

    body = functools.partial(_body, n, n_steps, p_sz, p_nsz)

    out = pl.pallas_call(
        body,
        grid=(n_steps,),
        in_specs=[
            pl.BlockSpec((_CHUNK, 1), lambda c: (c, 0)),
            pl.BlockSpec((_CHUNK, l), lambda c: (c, 0)),
            pl.BlockSpec((all_emb.shape[0], l), lambda c: (0, 0)),
            pl.BlockSpec(memory_space=pltpu.SMEM),
            pl.BlockSpec(memory_space=pltpu.SMEM),
        ],
        out_specs=pl.BlockSpec((1, 1), lambda c: (0, 0)),
        out_shape=jax.ShapeDtypeStruct((1, 1), jnp.float32),
        scratch_shapes=[
            pltpu.VMEM((16, l), jnp.float32),
            pltpu.SMEM((1,), jnp.float32),
        ],
    )(yf, h2, all_emb, Psz_idx, Pnsz_idx)
    return out[0, 0]


# speedup vs baseline: 1.4613x; 1.4125x over previous
import functools

import jax
import jax.numpy as jnp
from jax.experimental import pallas as pl
from jax.experimental.pallas import tpu as pltpu

_TEMPERATURE = 0.07
_CHUNK = 2048


def _body(n_rows, n_steps, p_sz, p_nsz,
          y_ref, h_ref, emb_ref, psz_ref, pnsz_ref,
          out_ref, acc_ref, cnt_ref):
    c = pl.program_id(0)

    @pl.when(c == 0)
    def _init():
        acc_ref[...] = jnp.zeros_like(acc_ref)
        cnt_ref[0] = jnp.float32(0.0)

    hb = h_ref[...]
    yb = y_ref[...]
    ym = yb.reshape(_CHUNK // 128, 128, 1) > 0.5
    sel = jnp.where(ym, hb.reshape(_CHUNK // 128, 128, 512), 0.0)
    h3 = hb.reshape(_CHUNK // 8, 8, 512)
    s3 = sel.reshape(_CHUNK // 8, 8, 512)
    acc_ref[0:8, :] += jnp.sum(h3, axis=0)
    acc_ref[8:16, :] += jnp.sum(s3, axis=0)
    cnt_ref[0] += jnp.sum(yb)

    @pl.when(c == n_steps - 1)
    def _epilogue():
        s_all = jnp.sum(acc_ref[0:8, :], axis=0, keepdims=True)
        s_y = jnp.sum(acc_ref[8:16, :], axis=0, keepdims=True)
        n1 = cnt_ref[0]
        n0 = jnp.float32(n_rows) - n1
        m_sz = s_y / jnp.maximum(n1, 1.0)
        m_nsz = (s_all - s_y) / jnp.maximum(n0, 1.0)
        m_sz = m_sz / jnp.maximum(jnp.sqrt(jnp.sum(m_sz * m_sz)), 1e-12)
        m_nsz = m_nsz / jnp.maximum(jnp.sqrt(jnp.sum(m_nsz * m_nsz)), 1e-12)

        emb = emb_ref[...]
        z_sz = jax.lax.dot_general(
            m_sz, emb, (((1,), (1,)), ((), ())),
            preferred_element_type=jnp.float32) / _TEMPERATURE
        z_nsz = jax.lax.dot_general(
            m_nsz, emb, (((1,), (1,)), ((), ())),
            preferred_element_type=jnp.float32) / _TEMPERATURE

        sim_sz = jnp.exp(z_sz)
        sim_nsz = jnp.exp(z_nsz)
        lanes = jax.lax.broadcasted_iota(jnp.int32, z_sz.shape, 1)

        mask_sz = jnp.zeros(z_sz.shape, dtype=jnp.bool_)
        zsum_sz = jnp.float32(0.0)
        for i in range(p_sz):
            hit = lanes == psz_ref[i]
            mask_sz = jnp.logical_or(mask_sz, hit)
            zsum_sz += jnp.sum(jnp.where(hit, z_sz, 0.0))
        den_sz = jnp.sum(jnp.where(mask_sz, 0.0, sim_sz))
        loss_sz = jnp.log(den_sz) - zsum_sz / jnp.float32(p_sz)

        mask_nsz = jnp.zeros(z_nsz.shape, dtype=jnp.bool_)
        zsum_nsz = jnp.float32(0.0)
        for i in range(p_nsz):
            hit = lanes == pnsz_ref[i]
            mask_nsz = jnp.logical_or(mask_nsz, hit)
            zsum_nsz += jnp.sum(jnp.where(hit, z_nsz, 0.0))
        den_nsz = jnp.sum(jnp.where(mask_nsz, 0.0, sim_nsz))
        loss_nsz = jnp.log(den_nsz) - zsum_nsz / jnp.float32(p_nsz)

        out_ref[...] = jnp.broadcast_to(loss_sz + loss_nsz, (1, 1))


def kernel(hg, all_emb, det_labels, concept_labels, Psz_idx, Pnsz_idx):
    del concept_labels
    b, nsz, t, l = hg.shape
    n = b * nsz * t
    h2 = hg.reshape(n, l)
    yf = det_labels.reshape(n // 128, 128).astype(jnp.float32)

    n_steps = n // _CHUNK
    p_sz = int(Psz_idx.shape[0])
    p_nsz = int(Pnsz_idx.shape[0])

    body = functools.partial(_body, n, n_steps, p_sz, p_nsz)

    out = pl.pallas_call(
        body,
        grid=(n_steps,),
        in_specs=[
            pl.BlockSpec((_CHUNK // 128, 128), lambda c: (c, 0)),
            pl.BlockSpec((_CHUNK, l), lambda c: (c, 0)),
            pl.BlockSpec((all_emb.shape[0], l), lambda c: (0, 0)),
            pl.BlockSpec(memory_space=pltpu.SMEM),
            pl.BlockSpec(memory_space=pltpu.SMEM),
        ],
        out_specs=pl.BlockSpec((1, 1), lambda c: (0, 0)),
        out_shape=jax.ShapeDtypeStruct((1, 1), jnp.float32),
        scratch_shapes=[
            pltpu.VMEM((16, l), jnp.float32),
            pltpu.SMEM((1,), jnp.float32),
        ],
    )(yf, h2, all_emb, Psz_idx, Pnsz_idx)
    return out[0, 0]
